# Initial kernel scaffold; baseline (speedup 1.0000x reference)
import jax
import jax.numpy as jnp
from jax.experimental import pallas as pl


def kernel(x, edge_index, edge_attr, batch, W_in, b_in, W_e, b_e, W_msg, b_msg, W_upd, b_upd, W_p1, b_p1, W_p2, b_p2):
    raise NotImplementedError("write your pallas kernel here")


# speedup vs baseline: 1.0226x; 1.0226x over previous
import functools

import jax
import jax.numpy as jnp
import numpy as np
from jax.experimental import pallas as pl

N = 10000
F = 128
E = 320000
D = 16
H = 64
P = 32
L = 4
B = 64
TEMP = 0.5

_rng1 = np.random.default_rng(1)
_KEEP1 = np.nonzero(_rng1.random(E) >= 0.2)[0]
_NODE_MASK = (_rng1.random(N) >= 0.1).astype(np.float32)
_rng2 = np.random.default_rng(2)
_KEEP2 = np.nonzero(_rng2.random(E) >= 0.2)[0]
_NOISE = (_rng2.normal(size=(N, F)) * 0.1).astype(np.float32)


def _proj_loss_kernel(g1_ref, g2_ref, wp1_ref, bp1_ref, wp2_ref, bp2_ref,
                      loss_ref, z1_ref, z2_ref):
    g1 = g1_ref[...]
    g2 = g2_ref[...]
    wp1 = wp1_ref[...]
    bp1 = bp1_ref[...]
    wp2 = wp2_ref[...]
    bp2 = bp2_ref[...]

    def proj(g):
        hid = jnp.maximum(jnp.dot(g, wp1, preferred_element_type=jnp.float32)
                          + bp1, 0.0)
        return jnp.dot(hid, wp2, preferred_element_type=jnp.float32) + bp2

    z1 = proj(g1)
    z2 = proj(g2)
    z1_ref[...] = z1
    z2_ref[...] = z2
    z1n = z1 / (jnp.sqrt(jnp.sum(z1 * z1, axis=1, keepdims=True)) + 1e-8)
    z2n = z2 / (jnp.sqrt(jnp.sum(z2 * z2, axis=1, keepdims=True)) + 1e-8)
    z = jnp.concatenate([z1n, z2n], axis=0)
    sim = jnp.dot(z, z.T, preferred_element_type=jnp.float32) / TEMP
    n2 = 2 * B
    eye = (jax.lax.broadcasted_iota(jnp.int32, (n2, n2), 0)
           == jax.lax.broadcasted_iota(jnp.int32, (n2, n2), 1))
    sim = jnp.where(eye, sim - 1e9, sim)
    row_max = jnp.max(sim, axis=1, keepdims=True)
    lse = jnp.log(jnp.sum(jnp.exp(sim - row_max), axis=1, keepdims=True)) + row_max
    rows = jax.lax.broadcasted_iota(jnp.int32, (n2, n2), 0)
    cols = jax.lax.broadcasted_iota(jnp.int32, (n2, n2), 1)
    pos_mask = ((rows + B) % n2) == cols
    pos = jnp.sum(jnp.where(pos_mask, sim, 0.0), axis=1, keepdims=True)
    loss_ref[...] = jnp.mean(lse - pos) * jnp.ones((1, 1), jnp.float32)


def _proj_loss(g1, g2, wp1, bp1, wp2, bp2):
    out_shapes = (
        jax.ShapeDtypeStruct((1, 1), jnp.float32),
        jax.ShapeDtypeStruct((B, P), jnp.float32),
        jax.ShapeDtypeStruct((B, P), jnp.float32),
    )
    loss, z1, z2 = pl.pallas_call(
        _proj_loss_kernel,
        out_shape=out_shapes,
    )(g1, g2, wp1, bp1.reshape(1, -1), wp2, bp2.reshape(1, -1))
    return loss[0, 0], z1, z2


def kernel(x, edge_index, edge_attr, batch, W_in, b_in, W_e, b_e,
           W_msg, b_msg, W_upd, b_upd, W_p1, b_p1, W_p2, b_p2):
    keep1 = jnp.asarray(_KEEP1)
    keep2 = jnp.asarray(_KEEP2)
    node_mask = jnp.asarray(_NODE_MASK)
    noise = jnp.asarray(_NOISE)

    x1 = x * node_mask[:, None]
    src1 = jnp.take(edge_index[0], keep1)
    dst1 = jnp.take(edge_index[1], keep1)
    ea1 = jnp.take(edge_attr, keep1, axis=0)
    x2 = x + noise
    src2 = jnp.take(edge_index[0], keep2)
    dst2 = jnp.take(edge_index[1], keep2)
    ea2 = jnp.take(edge_attr, keep2, axis=0)

    def encode(xv, src, dst, eav):
        h = jax.nn.relu(xv @ W_in + b_in)
        e = jax.nn.relu(eav @ W_e + b_e)
        for l in range(L):
            m = jax.nn.relu((h[src] + e) @ W_msg[l] + b_msg[l])
            agg = jax.ops.segment_sum(m, dst, num_segments=N)
            h = jax.nn.relu(h + agg @ W_upd[l] + b_upd[l])
        return h

    h1 = encode(x1, src1, dst1, ea1)
    h2 = encode(x2, src2, dst2, ea2)
    ones = jnp.ones((N,), jnp.float32)
    cnt = jnp.maximum(jax.ops.segment_sum(ones, batch, num_segments=B), 1.0)[:, None]
    g1 = jax.ops.segment_sum(h1, batch, num_segments=B) / cnt
    g2 = jax.ops.segment_sum(h2, batch, num_segments=B) / cnt

    loss, z1, z2 = _proj_loss(g1, g2, W_p1, b_p1, W_p2, b_p2)
    return (loss, z1, z2)


# speedup vs baseline: 8.9082x; 8.7111x over previous
import functools

import jax
import jax.numpy as jnp
import numpy as np
from jax import lax
from jax.experimental import pallas as pl
from jax.experimental.pallas import tpu as pltpu
from jax.experimental.pallas import tpu_sc as plsc

N = 10000
F = 128
E = 320000
D = 16
H = 64
P = 32
L = 4
B = 64
TEMP = 0.5

NP = 10240
TRASH = N
NC, NS = 2, 16
NW = NC * NS
CHUNK = 512
E_PAD = 327680
CPW = E_PAD // (NW * CHUNK)
RPT = NP // NS

_rng1 = np.random.default_rng(1)
_KEEP1 = np.asarray(_rng1.random(E) >= 0.2)
_NODE_MASK = (_rng1.random(N) >= 0.1).astype(np.float32)
_rng2 = np.random.default_rng(2)
_KEEP2 = np.asarray(_rng2.random(E) >= 0.2)
_NOISE = (_rng2.normal(size=(N, F)) * 0.1).astype(np.float32)


def _mp_body(hw_hbm, u_hbm, src_hbm, dst_hbm, zeros_hbm, out_hbm,
             idx_s_v, idx_d_v, u_v, rows_v, agg_sh, sem):
    c = lax.axis_index("c")
    s = lax.axis_index("s")
    wid = c * NS + s

    pltpu.sync_copy(zeros_hbm.at[pl.ds(s * RPT, RPT)],
                    agg_sh.at[pl.ds(s * RPT, RPT)])
    plsc.subcore_barrier()

    def chunk_step(k, carry):
        row_base = (wid * CPW + k) * (CHUNK // 128)
        ebase = (wid * CPW + k) * CHUNK
        pltpu.sync_copy(src_hbm.at[pl.ds(row_base, CHUNK // 128)], idx_s_v)
        pltpu.sync_copy(dst_hbm.at[pl.ds(row_base, CHUNK // 128)], idx_d_v)
        pltpu.sync_copy(u_hbm.at[pl.ds(ebase, CHUNK)], u_v)
        handles = [
            pltpu.async_copy(hw_hbm.at[idx_s_v.at[j]],
                             rows_v.at[pl.ds(j * 128, 128)], sem)
            for j in range(CHUNK // 128)
        ]
        for h in handles:
            h.wait()

        def row_step(i, carry2):
            for j in range(H // 16):
                sl = pl.ds(j * 16, 16)
                rows_v[i, sl] = jnp.maximum(rows_v[i, sl] + u_v[i, sl], 0.0)
            return carry2

        lax.fori_loop(0, CHUNK, row_step, 0, unroll=2)
        for j in range(CHUNK // 128):
            pltpu.sync_copy(rows_v.at[pl.ds(j * 128, 128)],
                            agg_sh.at[idx_d_v.at[j]], add=True)
        return carry

    lax.fori_loop(0, CPW, chunk_step, 0)
    plsc.subcore_barrier()
    pltpu.sync_copy(agg_sh.at[pl.ds(s * RPT, RPT)],
                    out_hbm.at[pl.ds(c * NP + s * RPT, RPT)])


_mp_call = functools.partial(
    pl.kernel,
    mesh=plsc.VectorSubcoreMesh(core_axis_name="c", subcore_axis_name="s"),
    out_type=jax.ShapeDtypeStruct((NC * NP, H), jnp.float32),
    compiler_params=pltpu.CompilerParams(use_tc_tiling_on_sc=False),
    scratch_types=[
        pltpu.VMEM((CHUNK // 128, 128), jnp.int32),
        pltpu.VMEM((CHUNK // 128, 128), jnp.int32),
        pltpu.VMEM((CHUNK, H), jnp.float32),
        pltpu.VMEM((CHUNK, H), jnp.float32),
        pltpu.VMEM_SHARED((NP, H), jnp.float32),
        pltpu.SemaphoreType.DMA,
    ],
)(_mp_body)


def _message_pass(hw, u, src2d, dst2d, zeros_np):
    return _mp_call(hw, u, src2d, dst2d, zeros_np)


def _proj_loss_kernel(gs1_ref, cb1_ref, gs2_ref, cb2_ref,
                      wp1_ref, bp1_ref, wp2_ref, bp2_ref,
                      loss_ref, z1_ref, z2_ref):
    cnt1 = jnp.maximum(cb1_ref[...][:, 0:1], 1.0)
    cnt2 = jnp.maximum(cb2_ref[...][:, 0:1], 1.0)
    g1 = gs1_ref[...] / cnt1
    g2 = gs2_ref[...] / cnt2
    wp1 = wp1_ref[...]
    bp1 = bp1_ref[...]
    wp2 = wp2_ref[...]
    bp2 = bp2_ref[...]

    def proj(g):
        hid = jnp.maximum(jnp.dot(g, wp1, preferred_element_type=jnp.float32)
                          + bp1, 0.0)
        return jnp.dot(hid, wp2, preferred_element_type=jnp.float32) + bp2

    z1 = proj(g1)
    z2 = proj(g2)
    z1_ref[...] = z1
    z2_ref[...] = z2
    z1n = z1 / (jnp.sqrt(jnp.sum(z1 * z1, axis=1, keepdims=True)) + 1e-8)
    z2n = z2 / (jnp.sqrt(jnp.sum(z2 * z2, axis=1, keepdims=True)) + 1e-8)
    z = jnp.concatenate([z1n, z2n], axis=0)
    sim = jnp.dot(z, z.T, preferred_element_type=jnp.float32) / TEMP
    n2 = 2 * B
    rows = jax.lax.broadcasted_iota(jnp.int32, (n2, n2), 0)
    cols = jax.lax.broadcasted_iota(jnp.int32, (n2, n2), 1)
    sim = jnp.where(rows == cols, sim - 1e9, sim)
    row_max = jnp.max(sim, axis=1, keepdims=True)
    lse = jnp.log(jnp.sum(jnp.exp(sim - row_max), axis=1, keepdims=True)) + row_max
    pos_mask = ((rows + B) % n2) == cols
    pos = jnp.sum(jnp.where(pos_mask, sim, 0.0), axis=1, keepdims=True)
    loss_ref[...] = jnp.mean(lse - pos) * jnp.ones((1, 1), jnp.float32)


def _proj_loss(gs1, cb1, gs2, cb2, wp1, bp1, wp2, bp2):
    out_shapes = (
        jax.ShapeDtypeStruct((1, 1), jnp.float32),
        jax.ShapeDtypeStruct((B, P), jnp.float32),
        jax.ShapeDtypeStruct((B, P), jnp.float32),
    )
    loss, z1, z2 = pl.pallas_call(
        _proj_loss_kernel,
        out_shape=out_shapes,
    )(gs1, cb1, gs2, cb2, wp1, bp1.reshape(1, -1), wp2, bp2.reshape(1, -1))
    return loss[0, 0], z1, z2


def kernel(x, edge_index, edge_attr, batch, W_in, b_in, W_e, b_e,
           W_msg, b_msg, W_upd, b_upd, W_p1, b_p1, W_p2, b_p2):
    keep1 = jnp.asarray(_KEEP1)
    keep2 = jnp.asarray(_KEEP2)
    node_mask = jnp.asarray(_NODE_MASK)
    noise = jnp.asarray(_NOISE)

    x1 = x * node_mask[:, None]
    x2 = x + noise
    src = edge_index[0]
    dst = edge_index[1]
    dst1 = jnp.where(keep1, dst, TRASH)
    dst2 = jnp.where(keep2, dst, TRASH)

    def pad_e(a, fill):
        return jnp.concatenate(
            [a, jnp.full((E_PAD - E,) + a.shape[1:], fill, a.dtype)], axis=0)

    src_p = pad_e(src, 0).reshape(E_PAD // 128, 128)
    dst1_p = pad_e(dst1, TRASH).reshape(E_PAD // 128, 128)
    dst2_p = pad_e(dst2, TRASH).reshape(E_PAD // 128, 128)
    ea_p = pad_e(edge_attr, 0.0)

    def pad_n(a):
        return jnp.concatenate(
            [a, jnp.zeros((NP - N,) + a.shape[1:], a.dtype)], axis=0)

    x1_p = pad_n(x1)
    x2_p = pad_n(x2)
    zeros_np = jnp.zeros((NP, H), jnp.float32)

    e_feat = jax.nn.relu(ea_p @ W_e + b_e)

    def encode(xv_p, dst_p):
        h = jax.nn.relu(xv_p @ W_in + b_in)
        for l in range(L):
            hw = h @ W_msg[l] + b_msg[l]
            u = e_feat @ W_msg[l]
            aggp = _message_pass(hw, u, src_p, dst_p, zeros_np)
            agg = aggp[:NP] + aggp[NP:]
            h = jax.nn.relu(h + agg @ W_upd[l] + b_upd[l])
        return h

    h1 = encode(x1_p, dst1_p)[:N]
    h2 = encode(x2_p, dst2_p)[:N]

    ones = jnp.ones((N,), jnp.float32)
    cnt = jnp.maximum(jax.ops.segment_sum(ones, batch, num_segments=B), 1.0)
    gs1 = jax.ops.segment_sum(h1, batch, num_segments=B)
    gs2 = jax.ops.segment_sum(h2, batch, num_segments=B)
    cb = jnp.broadcast_to(cnt[:, None], (B, 2 * H)).astype(jnp.float32)

    loss, z1, z2 = _proj_loss(gs1, cb, gs2, cb, W_p1, b_p1, W_p2, b_p2)
    return (loss, z1, z2)


# speedup vs baseline: 12.8991x; 1.4480x over previous
import functools

import jax
import jax.numpy as jnp
import numpy as np
from jax import lax
from jax.experimental import pallas as pl
from jax.experimental.pallas import tpu as pltpu
from jax.experimental.pallas import tpu_sc as plsc

N = 10000
F = 128
E = 320000
D = 16
H = 64
P = 32
L = 4
B = 64
TEMP = 0.5

NP = 10240
TRASH = N
NC, NS = 2, 16
NW = NC * NS
CHUNK = 256
E_PAD = 327680
EPW = E_PAD // NW
CPW = EPW // CHUNK
IDXR = CHUNK // 128
RPT = NP // NS
NBLK = NP // 256

_rng1 = np.random.default_rng(1)
_KEEP1 = np.asarray(_rng1.random(E) >= 0.2)
_NODE_MASK = (_rng1.random(N) >= 0.1).astype(np.float32)
_rng2 = np.random.default_rng(2)
_KEEP2 = np.asarray(_rng2.random(E) >= 0.2)
_NOISE = (_rng2.normal(size=(N, F)) * 0.1).astype(np.float32)

_F32 = jnp.float32


def _make_mp(l):
    def body(hw_hbm, u_hbm, src_hbm, d1_hbm, d2_hbm, z_hbm, out_hbm,
             is0, id0, u0, r0, is1, id1, u1, r1,
             agg, sem_ld, sem_g):
        idx_s = (is0, is1)
        idx_d = (id0, id1)
        u_v = (u0, u1)
        rows = (r0, r1)

        c = lax.axis_index("c")
        s = lax.axis_index("s")
        wid = c * NS + s
        row0 = wid * (EPW // 128)
        eb0 = l * E_PAD + wid * EPW

        def view_pass(v, d_hbm):
            pltpu.sync_copy(z_hbm.at[pl.ds(s * RPT, RPT)],
                            agg.at[pl.ds(s * RPT, RPT)])
            plsc.subcore_barrier()

            def fire_loads(q, b):
                rb = row0 + q * IDXR
                eb = eb0 + q * CHUNK
                pltpu.async_copy(src_hbm.at[pl.ds(rb, IDXR)], idx_s[b],
                                 sem_ld)
                pltpu.async_copy(d_hbm.at[pl.ds(rb, IDXR)], idx_d[b], sem_ld)
                pltpu.async_copy(u_hbm.at[pl.ds(eb, CHUNK)], u_v[b], sem_ld)

            def wait_loads(b):
                pltpu.make_async_copy(src_hbm.at[pl.ds(0, IDXR)], idx_s[b],
                                      sem_ld).wait()
                pltpu.make_async_copy(d_hbm.at[pl.ds(0, IDXR)], idx_d[b],
                                      sem_ld).wait()
                pltpu.make_async_copy(u_hbm.at[pl.ds(0, CHUNK)], u_v[b],
                                      sem_ld).wait()

            def fire_gathers(b):
                if v == 1:
                    for j in range(IDXR):
                        for t in range(8):
                            sl = pl.ds(t * 16, 16)
                            idx_s[b][j, sl] = idx_s[b][j, sl] + NP
                for j in range(IDXR):
                    pltpu.async_copy(hw_hbm.at[idx_s[b].at[j]],
                                     rows[b].at[pl.ds(j * 128, 128)], sem_g)

            def complete(b):
                for j in range(IDXR):
                    pltpu.make_async_copy(hw_hbm.at[pl.ds(0, 128)],
                                          rows[b].at[pl.ds(j * 128, 128)],
                                          sem_g).wait()

                @pl.loop(0, CHUNK)
                def _row(i):
                    for t in range(H // 16):
                        sl = pl.ds(t * 16, 16)
                        rows[b][i, sl] = jnp.maximum(
                            rows[b][i, sl] + u_v[b][i, sl], 0.0)

                for j in range(IDXR):
                    pltpu.sync_copy(rows[b].at[pl.ds(j * 128, 128)],
                                    agg.at[idx_d[b].at[j]], add=True)

            fire_loads(0, 0)

            @pl.loop(0, CPW, step=2)
            def _outer(q0):
                for b in (0, 1):
                    q = q0 + b
                    wait_loads(b)
                    fire_gathers(b)

                    @pl.when(q > 0)
                    def _():
                        complete(1 - b)

                    @pl.when(q < CPW - 1)
                    def _():
                        fire_loads(q + 1, 1 - b)

            complete((CPW - 1) % 2)
            plsc.subcore_barrier()
            pltpu.sync_copy(
                agg.at[pl.ds(s * RPT, RPT)],
                out_hbm.at[pl.ds((v * NC + c) * NP + s * RPT, RPT)])
            plsc.subcore_barrier()

        view_pass(0, d1_hbm)
        view_pass(1, d2_hbm)

    buf = lambda: [
        pltpu.VMEM((IDXR, 128), jnp.int32),
        pltpu.VMEM((IDXR, 128), jnp.int32),
        pltpu.VMEM((CHUNK, H), _F32),
        pltpu.VMEM((CHUNK, H), _F32),
    ]
    return functools.partial(
        pl.kernel,
        mesh=plsc.VectorSubcoreMesh(core_axis_name="c", subcore_axis_name="s"),
        out_type=jax.ShapeDtypeStruct((2 * NC * NP, H), _F32),
        compiler_params=pltpu.CompilerParams(use_tc_tiling_on_sc=False),
        scratch_types=buf() + buf() + [
            pltpu.VMEM_SHARED((NP, H), _F32),
            pltpu.SemaphoreType.DMA,
            pltpu.SemaphoreType.DMA,
        ],
    )(body)


_MP = [_make_mp(l) for l in range(L)]


def _dot(a, b):
    return jnp.dot(a, b, preferred_element_type=_F32)


def _t1_body(x_ref, win_ref, bin_ref, wm_ref, bm_ref, h_ref, hw_ref):
    h = jnp.maximum(_dot(x_ref[0], win_ref[...]) + bin_ref[...], 0.0)
    h_ref[0] = h
    hw_ref[0] = _dot(h, wm_ref[...]) + bm_ref[...]


def _t1(x12, W_in, b_in, W_msg0, b_msg0):
    return pl.pallas_call(
        _t1_body,
        grid=(2, NBLK),
        in_specs=[
            pl.BlockSpec((1, 256, F), lambda v, i: (v, i, 0)),
            pl.BlockSpec((F, H), lambda v, i: (0, 0)),
            pl.BlockSpec((1, H), lambda v, i: (0, 0)),
            pl.BlockSpec((H, H), lambda v, i: (0, 0)),
            pl.BlockSpec((1, H), lambda v, i: (0, 0)),
        ],
        out_specs=[
            pl.BlockSpec((1, 256, H), lambda v, i: (v, i, 0)),
            pl.BlockSpec((1, 256, H), lambda v, i: (v, i, 0)),
        ],
        out_shape=[
            jax.ShapeDtypeStruct((2, NP, H), _F32),
            jax.ShapeDtypeStruct((2, NP, H), _F32),
        ],
    )(x12, W_in, b_in.reshape(1, H), W_msg0, b_msg0.reshape(1, H))


_T2_BLK = 2048


def _t2_body(ea_ref, we_ref, be_ref, wm_ref, u_ref):
    e = jnp.maximum(_dot(ea_ref[...], we_ref[...]) + be_ref[...], 0.0)
    for l in range(L):
        u_ref[l] = _dot(e, wm_ref[l])


def _t2(ea_p, W_e, b_e, W_msg):
    return pl.pallas_call(
        _t2_body,
        grid=(E_PAD // _T2_BLK,),
        in_specs=[
            pl.BlockSpec((_T2_BLK, D), lambda i: (i, 0)),
            pl.BlockSpec((D, H), lambda i: (0, 0)),
            pl.BlockSpec((1, H), lambda i: (0, 0)),
            pl.BlockSpec((L, H, H), lambda i: (0, 0, 0)),
        ],
        out_specs=pl.BlockSpec((L, _T2_BLK, H), lambda i: (0, i, 0)),
        out_shape=jax.ShapeDtypeStruct((L, E_PAD, H), _F32),
    )(ea_p, W_e, b_e.reshape(1, H), W_msg)


def _t3_body(has_next, h_ref, agg_ref, wu_ref, bu_ref, wm_ref, bm_ref,
             h2_ref, hw2_ref=None):
    agg = agg_ref[0, 0] + agg_ref[0, 1]
    hn = jnp.maximum(h_ref[0] + _dot(agg, wu_ref[...]) + bu_ref[...], 0.0)
    h2_ref[0] = hn
    if has_next:
        hw2_ref[0] = _dot(hn, wm_ref[...]) + bm_ref[...]


def _t3(h2, aggp, wu, bu, wm, bm, has_next):
    nout = 2 if has_next else 1
    out = pl.pallas_call(
        functools.partial(_t3_body, has_next),
        grid=(2, NBLK),
        in_specs=[
            pl.BlockSpec((1, 256, H), lambda v, i: (v, i, 0)),
            pl.BlockSpec((1, 2, 256, H), lambda v, i: (v, 0, i, 0)),
            pl.BlockSpec((H, H), lambda v, i: (0, 0)),
            pl.BlockSpec((1, H), lambda v, i: (0, 0)),
            pl.BlockSpec((H, H), lambda v, i: (0, 0)),
            pl.BlockSpec((1, H), lambda v, i: (0, 0)),
        ],
        out_specs=[pl.BlockSpec((1, 256, H), lambda v, i: (v, i, 0))] * nout,
        out_shape=[jax.ShapeDtypeStruct((2, NP, H), _F32)] * nout,
    )(h2, aggp, wu, bu.reshape(1, H), wm, bm.reshape(1, H))
    return out if has_next else (out[0], None)


def _t4_body(h_ref, b_ref, gs_ref, cb_ref):
    v = pl.program_id(0)
    i = pl.program_id(1)
    bi = b_ref[0]
    oh = (jax.lax.broadcasted_iota(jnp.int32, (B, 256), 0)
          == jnp.broadcast_to(bi, (B, 256))).astype(_F32)
    part = _dot(oh, h_ref[0])

    @pl.when(i == 0)
    def _():
        gs_ref[0] = part

    @pl.when(i != 0)
    def _():
        gs_ref[0] += part

    cpart = _dot(oh, jnp.ones((256, 2 * H), _F32))

    @pl.when(jnp.logical_and(v == 0, i == 0))
    def _():
        cb_ref[...] = cpart

    @pl.when(jnp.logical_and(v == 0, i != 0))
    def _():
        cb_ref[...] += cpart


def _t4(h2, batch3d):
    return pl.pallas_call(
        _t4_body,
        grid=(2, NBLK),
        in_specs=[
            pl.BlockSpec((1, 256, H), lambda v, i: (v, i, 0)),
            pl.BlockSpec((1, 1, 256), lambda v, i: (i, 0, 0)),
        ],
        out_specs=[
            pl.BlockSpec((1, B, H), lambda v, i: (v, 0, 0)),
            pl.BlockSpec((B, 2 * H), lambda v, i: (0, 0)),
        ],
        out_shape=[
            jax.ShapeDtypeStruct((2, B, H), _F32),
            jax.ShapeDtypeStruct((B, 2 * H), _F32),
        ],
    )(h2, batch3d)


def _t5_body(gs1_ref, gs2_ref, cb_ref, wp1_ref, bp1_ref, wp2_ref, bp2_ref,
             loss_ref, z1_ref, z2_ref):
    cnt = jnp.maximum(cb_ref[...][:, 0:1], 1.0)
    g1 = gs1_ref[0] / cnt
    g2 = gs2_ref[0] / cnt
    wp1 = wp1_ref[...]
    bp1 = bp1_ref[...]
    wp2 = wp2_ref[...]
    bp2 = bp2_ref[...]

    def proj(g):
        hid = jnp.maximum(_dot(g, wp1) + bp1, 0.0)
        return _dot(hid, wp2) + bp2

    z1 = proj(g1)
    z2 = proj(g2)
    z1_ref[...] = z1
    z2_ref[...] = z2
    z1n = z1 / (jnp.sqrt(jnp.sum(z1 * z1, axis=1, keepdims=True)) + 1e-8)
    z2n = z2 / (jnp.sqrt(jnp.sum(z2 * z2, axis=1, keepdims=True)) + 1e-8)
    z = jnp.concatenate([z1n, z2n], axis=0)
    sim = _dot(z, z.T) / TEMP
    n2 = 2 * B
    rows = jax.lax.broadcasted_iota(jnp.int32, (n2, n2), 0)
    cols = jax.lax.broadcasted_iota(jnp.int32, (n2, n2), 1)
    sim = jnp.where(rows == cols, sim - 1e9, sim)
    row_max = jnp.max(sim, axis=1, keepdims=True)
    lse = jnp.log(jnp.sum(jnp.exp(sim - row_max), axis=1, keepdims=True)) + row_max
    pos_mask = ((rows + B) % n2) == cols
    pos = jnp.sum(jnp.where(pos_mask, sim, 0.0), axis=1, keepdims=True)
    loss_ref[...] = jnp.mean(lse - pos) * jnp.ones((1, 1), _F32)


def _t5(gs2, cb, wp1, bp1, wp2, bp2):
    out_shapes = (
        jax.ShapeDtypeStruct((1, 1), _F32),
        jax.ShapeDtypeStruct((B, P), _F32),
        jax.ShapeDtypeStruct((B, P), _F32),
    )
    gsplit = gs2.reshape(2, 1, B, H)
    loss, z1, z2 = pl.pallas_call(
        _t5_body,
        out_shape=out_shapes,
    )(gsplit[0], gsplit[1], cb, wp1, bp1.reshape(1, 2 * H),
      wp2, bp2.reshape(1, P))
    return loss[0, 0], z1, z2


def kernel(x, edge_index, edge_attr, batch, W_in, b_in, W_e, b_e,
           W_msg, b_msg, W_upd, b_upd, W_p1, b_p1, W_p2, b_p2):
    keep1 = jnp.asarray(_KEEP1)
    keep2 = jnp.asarray(_KEEP2)
    node_mask = jnp.asarray(_NODE_MASK)
    noise = jnp.asarray(_NOISE)

    def pad_n(a, fill=0):
        return jnp.concatenate(
            [a, jnp.full((NP - N,) + a.shape[1:], fill, a.dtype)], axis=0)

    def pad_e(a, fill):
        return jnp.concatenate(
            [a, jnp.full((E_PAD - E,) + a.shape[1:], fill, a.dtype)], axis=0)

    x1 = pad_n(x * node_mask[:, None])
    x2 = pad_n(x + noise)
    x12 = jnp.stack([x1, x2])

    src = edge_index[0]
    dst = edge_index[1]
    src2d = pad_e(src, 0).reshape(E_PAD // 128, 128)
    d1_2d = pad_e(jnp.where(keep1, dst, TRASH), TRASH).reshape(E_PAD // 128, 128)
    d2_2d = pad_e(jnp.where(keep2, dst, TRASH), TRASH).reshape(E_PAD // 128, 128)
    ea_p = pad_e(edge_attr, 0.0)
    batch3d = pad_n(batch, 127).reshape(NBLK, 1, 256)
    zeros_np = jnp.zeros((NP, H), _F32)

    h2, hw2 = _t1(x12, W_in, b_in, W_msg[0], b_msg[0])
    u4 = _t2(ea_p, W_e, b_e, W_msg).reshape(L * E_PAD, H)

    for l in range(L):
        aggp = _MP[l](hw2.reshape(2 * NP, H), u4, src2d, d1_2d, d2_2d,
                      zeros_np).reshape(2, 2, NP, H)
        has_next = l < L - 1
        wm = W_msg[l + 1] if has_next else W_msg[0]
        bm = b_msg[l + 1] if has_next else b_msg[0]
        h2, hw2 = _t3(h2, aggp, W_upd[l], b_upd[l], wm, bm, has_next)

    gs2, cb = _t4(h2, batch3d)
    loss, z1, z2 = _t5(gs2, cb, W_p1, b_p1, W_p2, b_p2)
    return (loss, z1, z2)
